# baseline (speedup 1.0000x reference)
import functools

import jax
import jax.numpy as jnp
from jax.experimental import pallas as pl
from jax.experimental.pallas import tpu as pltpu


def _pick_block(m, cap=2048):
    best = 8
    for e in range(8, cap + 1, 8):
        if m % e == 0:
            best = e
    return best


def _ln(h, g, z):
    mu = jnp.mean(h, axis=-1, keepdims=True)
    v = jnp.mean((h - mu) ** 2, axis=-1, keepdims=True)
    return (h - mu) * jax.lax.rsqrt(v + 1e-5) * g + z


def _edge_mlp_body(xg_ref, ea_ref, s_ref,
                   wd1x_ref, wd1e_ref, bd1_ref, gd1_ref, zd1_ref,
                   wd2_ref, bd2_ref, gd2_ref, zd2_ref,
                   wt1x_ref, wt1e_ref, bt1_ref, gt1_ref, zt1_ref,
                   wt2_ref, bt2_ref, gt2_ref, zt2_ref,
                   we_ref, ce_ref,
                   cflow_ref, cnode_ref):
    xg = xg_ref[...]
    ea = ea_ref[...]
    s = s_ref[...]
    md = s[:, 0:1]
    mt = s[:, 1:2]
    wnode = s[:, 2:3]

    xgb = xg.astype(jnp.bfloat16)
    eab = ea.astype(jnp.bfloat16)

    def mlp(w1x, w1e, b1, g1, z1, w2, b2, g2, z2):
        h = (jnp.dot(xgb, w1x.astype(jnp.bfloat16),
                     preferred_element_type=jnp.float32)
             + jnp.dot(eab, w1e.astype(jnp.bfloat16),
                       preferred_element_type=jnp.float32) + b1)
        h = jax.nn.relu(_ln(h, g1, z1))
        h = jnp.dot(h.astype(jnp.bfloat16), w2.astype(jnp.bfloat16),
                    preferred_element_type=jnp.float32) + b2
        return jax.nn.relu(_ln(h, g2, z2))

    fd = mlp(wd1x_ref[...], wd1e_ref[...], bd1_ref[...], gd1_ref[...],
             zd1_ref[...], wd2_ref[...], bd2_ref[...], gd2_ref[...],
             zd2_ref[...])
    ft = mlp(wt1x_ref[...], wt1e_ref[...], bt1_ref[...], gt1_ref[...],
             zt1_ref[...], wt2_ref[...], bt2_ref[...], gt2_ref[...],
             zt2_ref[...])

    ew = jax.nn.sigmoid(
        jnp.dot(ea, we_ref[...], preferred_element_type=jnp.float32)
        + ce_ref[...])
    cflow_ref[...] = ew * (md * fd + mt * ft)
    cnode_ref[...] = wnode * xg


def _edge_mlp(xg, ea, s, wd1x, wd1e, bd1, gd1, zd1, wd2, bd2, gd2, zd2,
              wt1x, wt1e, bt1, gt1, zt1, wt2, bt2, gt2, zt2, we, ce):
    m, df = xg.shape
    de = ea.shape[1]
    dh = wd1x.shape[1]
    E = _pick_block(m)
    grid = (m // E,)
    edge_spec = lambda c: pl.BlockSpec((E, c), lambda g: (g, 0))
    w_spec = lambda r, c: pl.BlockSpec((r, c), lambda g: (0, 0))
    return pl.pallas_call(
        _edge_mlp_body,
        grid=grid,
        in_specs=[
            edge_spec(df), edge_spec(de), edge_spec(4),
            w_spec(df, dh), w_spec(de, dh), w_spec(1, dh), w_spec(1, dh),
            w_spec(1, dh),
            w_spec(dh, df), w_spec(1, df), w_spec(1, df), w_spec(1, df),
            w_spec(df, dh), w_spec(de, dh), w_spec(1, dh), w_spec(1, dh),
            w_spec(1, dh),
            w_spec(dh, df), w_spec(1, df), w_spec(1, df), w_spec(1, df),
            w_spec(de, 1), w_spec(1, 1),
        ],
        out_specs=[edge_spec(df), edge_spec(df)],
        out_shape=[
            jax.ShapeDtypeStruct((m, df), jnp.float32),
            jax.ShapeDtypeStruct((m, df), jnp.float32),
        ],
    )(xg, ea, s, wd1x, wd1e, bd1, gd1, zd1, wd2, bd2, gd2, zd2,
      wt1x, wt1e, bt1, gt1, zt1, wt2, bt2, gt2, zt2, we, ce)


def _fuse_body(flow_ref, node_ref, wn_ref, cn_ref, wm_ref, cm_ref,
               gm_ref, zm_ref, out_ref):
    flow = flow_ref[...]
    node = node_ref[...]
    nw = jax.nn.sigmoid(
        jnp.dot(flow, wn_ref[...], preferred_element_type=jnp.float32)
        + cn_ref[...])
    fused = flow + nw * node
    h = jnp.dot(fused, wm_ref[...], preferred_element_type=jnp.float32) \
        + cm_ref[...]
    out_ref[...] = jax.nn.relu(_ln(h, gm_ref[...], zm_ref[...]))


def _fuse(flow, node, wn, cn, wm, cm, gm, zm):
    n, df = flow.shape
    NB = _pick_block(n)
    grid = (n // NB,)
    node_spec = pl.BlockSpec((NB, df), lambda g: (g, 0))
    w_spec = lambda r, c: pl.BlockSpec((r, c), lambda g: (0, 0))
    return pl.pallas_call(
        _fuse_body,
        grid=grid,
        in_specs=[
            node_spec, node_spec,
            w_spec(df, 1), w_spec(1, 1), w_spec(df, df), w_spec(1, df),
            w_spec(1, df), w_spec(1, df),
        ],
        out_specs=node_spec,
        out_shape=jax.ShapeDtypeStruct((n, df), jnp.float32),
    )(flow, node, wn, cn, wm, cm, gm, zm)


def kernel(x, edge_index, edge_attr, Wd1, bd1, gd1, zd1, Wd2, bd2, gd2, zd2,
           Wt1, bt1, gt1, zt1, Wt2, bt2, gt2, zt2, We, ce, Wn, cn, Wm, cm,
           gm, zm):
    n, df = x.shape
    row = edge_index[0]
    col = edge_index[1]
    m = row.shape[0]

    xg = jnp.take(x, col, axis=0)

    inv = 1.0 / jnp.clip(jnp.sqrt(jnp.sum(x * x, axis=1)), 1e-12)
    sim = jnp.sum(xg * jnp.take(x, row, axis=0), axis=1) \
        * jnp.take(inv, row) * jnp.take(inv, col)

    mask_d = row < col
    mask_t = row > col
    md = mask_d.astype(jnp.float32)
    mt = mask_t.astype(jnp.float32)

    rank_d = jnp.cumsum(mask_d.astype(jnp.int32)) - 1
    rank_t = jnp.cumsum(mask_t.astype(jnp.int32)) - 1
    s_d = jnp.zeros((m,), jnp.float32).at[
        jnp.where(mask_d, rank_d, m)].set(sim, mode='drop')
    sim_t = jnp.take(s_d, jnp.clip(rank_t, 0, m - 1))
    wnode = sim * md + sim_t * mt

    s = jnp.stack([md, mt, wnode, jnp.zeros_like(md)], axis=1)

    dh = Wd1.shape[1]
    de = edge_attr.shape[1]
    r2 = lambda v: v.reshape(1, -1)
    cflow, cnode = _edge_mlp(
        xg, edge_attr, s,
        Wd1[:df], Wd1[df:], r2(bd1), r2(gd1), r2(zd1),
        Wd2, r2(bd2), r2(gd2), r2(zd2),
        Wt1[:df], Wt1[df:], r2(bt1), r2(gt1), r2(zt1),
        Wt2, r2(bt2), r2(gt2), r2(zt2),
        We, r2(ce))

    agg = jax.ops.segment_sum(
        jnp.concatenate([cflow, cnode], axis=1), row, num_segments=n)
    flow_total = agg[:, :df]
    node_total = agg[:, df:]

    return _fuse(flow_total, node_total, Wn, r2(cn), Wm, r2(cm),
                 r2(gm), r2(zm))


# speedup vs baseline: 1.5668x; 1.5668x over previous
import functools

import jax
import jax.numpy as jnp
from jax.experimental import pallas as pl
from jax.experimental.pallas import tpu as pltpu


def _pick_block(m, cap=2048):
    best = 8
    for e in range(8, cap + 1, 8):
        if m % e == 0:
            best = e
    return best


def _ln(h, g, z):
    mu = jnp.mean(h, axis=-1, keepdims=True)
    v = jnp.mean((h - mu) ** 2, axis=-1, keepdims=True)
    return (h - mu) * jax.lax.rsqrt(v + 1e-5) * g + z


def _edge_mlp_body(xg_ref, ea_ref, s_ref,
                   wd1x_ref, wd1e_ref, bd1_ref, gd1_ref, zd1_ref,
                   wd2_ref, bd2_ref, gd2_ref, zd2_ref,
                   wt1x_ref, wt1e_ref, bt1_ref, gt1_ref, zt1_ref,
                   wt2_ref, bt2_ref, gt2_ref, zt2_ref,
                   we_ref, ce_ref,
                   cflow_ref, cnode_ref):
    xg = xg_ref[...]
    ea = ea_ref[...]
    s = s_ref[...]
    md = s[:, 0:1]
    mt = s[:, 1:2]
    wnode = s[:, 2:3]

    xgb = xg.astype(jnp.bfloat16)
    eab = ea.astype(jnp.bfloat16)

    def mlp(w1x, w1e, b1, g1, z1, w2, b2, g2, z2):
        h = (jnp.dot(xgb, w1x.astype(jnp.bfloat16),
                     preferred_element_type=jnp.float32)
             + jnp.dot(eab, w1e.astype(jnp.bfloat16),
                       preferred_element_type=jnp.float32) + b1)
        h = jax.nn.relu(_ln(h, g1, z1))
        h = jnp.dot(h.astype(jnp.bfloat16), w2.astype(jnp.bfloat16),
                    preferred_element_type=jnp.float32) + b2
        return jax.nn.relu(_ln(h, g2, z2))

    fd = mlp(wd1x_ref[...], wd1e_ref[...], bd1_ref[...], gd1_ref[...],
             zd1_ref[...], wd2_ref[...], bd2_ref[...], gd2_ref[...],
             zd2_ref[...])
    ft = mlp(wt1x_ref[...], wt1e_ref[...], bt1_ref[...], gt1_ref[...],
             zt1_ref[...], wt2_ref[...], bt2_ref[...], gt2_ref[...],
             zt2_ref[...])

    ew = jax.nn.sigmoid(
        jnp.dot(ea, we_ref[...], preferred_element_type=jnp.float32)
        + ce_ref[...])
    cflow_ref[...] = ew * (md * fd + mt * ft)
    cnode_ref[...] = wnode * xg


def _edge_mlp(xg, ea, s, wd1x, wd1e, bd1, gd1, zd1, wd2, bd2, gd2, zd2,
              wt1x, wt1e, bt1, gt1, zt1, wt2, bt2, gt2, zt2, we, ce):
    m, df = xg.shape
    de = ea.shape[1]
    dh = wd1x.shape[1]
    E = _pick_block(m)
    grid = (m // E,)
    edge_spec = lambda c: pl.BlockSpec((E, c), lambda g: (g, 0))
    w_spec = lambda r, c: pl.BlockSpec((r, c), lambda g: (0, 0))
    return pl.pallas_call(
        _edge_mlp_body,
        grid=grid,
        in_specs=[
            edge_spec(df), edge_spec(de), edge_spec(4),
            w_spec(df, dh), w_spec(de, dh), w_spec(1, dh), w_spec(1, dh),
            w_spec(1, dh),
            w_spec(dh, df), w_spec(1, df), w_spec(1, df), w_spec(1, df),
            w_spec(df, dh), w_spec(de, dh), w_spec(1, dh), w_spec(1, dh),
            w_spec(1, dh),
            w_spec(dh, df), w_spec(1, df), w_spec(1, df), w_spec(1, df),
            w_spec(de, 1), w_spec(1, 1),
        ],
        out_specs=[edge_spec(df), edge_spec(df)],
        out_shape=[
            jax.ShapeDtypeStruct((m, df), jnp.float32),
            jax.ShapeDtypeStruct((m, df), jnp.float32),
        ],
    )(xg, ea, s, wd1x, wd1e, bd1, gd1, zd1, wd2, bd2, gd2, zd2,
      wt1x, wt1e, bt1, gt1, zt1, wt2, bt2, gt2, zt2, we, ce)


def _fuse_body(flow_ref, node_ref, wn_ref, cn_ref, wm_ref, cm_ref,
               gm_ref, zm_ref, out_ref):
    flow = flow_ref[...]
    node = node_ref[...]
    nw = jax.nn.sigmoid(
        jnp.dot(flow, wn_ref[...], preferred_element_type=jnp.float32)
        + cn_ref[...])
    fused = flow + nw * node
    h = jnp.dot(fused, wm_ref[...], preferred_element_type=jnp.float32) \
        + cm_ref[...]
    out_ref[...] = jax.nn.relu(_ln(h, gm_ref[...], zm_ref[...]))


def _fuse(flow, node, wn, cn, wm, cm, gm, zm):
    n, df = flow.shape
    NB = _pick_block(n)
    grid = (n // NB,)
    node_spec = pl.BlockSpec((NB, df), lambda g: (g, 0))
    w_spec = lambda r, c: pl.BlockSpec((r, c), lambda g: (0, 0))
    return pl.pallas_call(
        _fuse_body,
        grid=grid,
        in_specs=[
            node_spec, node_spec,
            w_spec(df, 1), w_spec(1, 1), w_spec(df, df), w_spec(1, df),
            w_spec(1, df), w_spec(1, df),
        ],
        out_specs=node_spec,
        out_shape=jax.ShapeDtypeStruct((n, df), jnp.float32),
    )(flow, node, wn, cn, wm, cm, gm, zm)


def kernel(x, edge_index, edge_attr, Wd1, bd1, gd1, zd1, Wd2, bd2, gd2, zd2,
           Wt1, bt1, gt1, zt1, Wt2, bt2, gt2, zt2, We, ce, Wn, cn, Wm, cm,
           gm, zm):
    n, df = x.shape
    row = edge_index[0]
    col = edge_index[1]
    m = row.shape[0]

    half = m // 2
    inv = 1.0 / jnp.clip(jnp.sqrt(jnp.sum(x * x, axis=1)), 1e-12)
    tab = jnp.concatenate([x, inv[:, None]], axis=1)
    tg = jnp.take(tab, col, axis=0)
    xg = tg[:, :df]
    invc = tg[:, df]

    dot_h1 = jnp.sum(xg[:half] * xg[half:], axis=1)
    invr = jnp.concatenate([invc[half:], invc[:half]])
    sim = jnp.concatenate([dot_h1, dot_h1]) * invc * invr

    mask_d = row < col
    mask_t = row > col
    md = mask_d.astype(jnp.float32)
    mt = mask_t.astype(jnp.float32)

    rank_d = jnp.cumsum(mask_d.astype(jnp.int32)) - 1
    rank_t = jnp.cumsum(mask_t.astype(jnp.int32)) - 1
    s_d = jnp.zeros((m,), jnp.float32).at[
        jnp.where(mask_d, rank_d, m)].set(sim, mode='drop')
    sim_t = jnp.take(s_d, jnp.clip(rank_t, 0, m - 1))
    wnode = sim * md + sim_t * mt

    s = jnp.stack([md, mt, wnode, jnp.zeros_like(md)], axis=1)

    dh = Wd1.shape[1]
    de = edge_attr.shape[1]
    r2 = lambda v: v.reshape(1, -1)
    cflow, cnode = _edge_mlp(
        xg, edge_attr, s,
        Wd1[:df], Wd1[df:], r2(bd1), r2(gd1), r2(zd1),
        Wd2, r2(bd2), r2(gd2), r2(zd2),
        Wt1[:df], Wt1[df:], r2(bt1), r2(gt1), r2(zt1),
        Wt2, r2(bt2), r2(gt2), r2(zt2),
        We, r2(ce))

    agg = jax.ops.segment_sum(
        jnp.concatenate([cflow, cnode], axis=1), row, num_segments=n)
    flow_total = agg[:, :df]
    node_total = agg[:, df:]

    return _fuse(flow_total, node_total, Wn, r2(cn), Wm, r2(cm),
                 r2(gm), r2(zm))
